# baseline (speedup 1.0000x reference)
import functools

import jax
import jax.numpy as jnp
from jax.experimental import pallas as pl
from jax.experimental.pallas import tpu as pltpu

T = 2048
C = 768
H = 12
HD = C // H
R = 256
NB1 = 4
W1 = NB1 * R
NEG = -1e30
SCALE = 1.0 / (HD ** 0.5)
BUDGET = max(1, int(T * 0.2))
KCAP = BUDGET - 1
BIG = 1e30
BIG_BITS = 0x7149F2CA


def _qkv_kernel(x_ref, w_ref, b_ref, o_ref):
    o_ref[...] = (
        jnp.dot(
            x_ref[...].astype(jnp.bfloat16),
            w_ref[...].astype(jnp.bfloat16),
            preferred_element_type=jnp.float32,
        )
        + b_ref[...]
    )


def _attn_body(width, row_off, q_ref, k_ref, v_ref, wp_ref, bp_ref, *rest):
    if row_off == 0:
        o_ref, accout_ref, acc_ref = rest
        accin_ref = None
    else:
        accin_ref, o_ref, acc_ref = rest
        accout_ref = None

    pid = pl.program_id(0)

    @pl.when(pid == 0)
    def _init():
        if accin_ref is None:
            acc_ref[...] = jnp.zeros_like(acc_ref)
        else:
            acc_ref[...] = jnp.concatenate(
                [accin_ref[...], jnp.zeros((8, width - W1), jnp.float32)], axis=1
            )

    i0 = (pid + row_off) * R
    rows = i0 + jax.lax.broadcasted_iota(jnp.int32, (R, 1), 0)
    cols = jax.lax.broadcasted_iota(jnp.int32, (R, width), 1)

    q0 = q_ref[:, 0:HD]
    k0 = k_ref[:, 0:HD]
    att0 = jax.lax.dot_general(
        q0, k0, (((1,), (1,)), ((), ())), preferred_element_type=jnp.float32
    ) * SCALE
    s_mask = (cols < rows) & (cols > 0)
    S = jnp.where(s_mask, jnp.maximum(att0, 0.0), 0.0)

    ri = jax.lax.broadcasted_iota(jnp.int32, (R, R), 0)
    rj = jax.lax.broadcasted_iota(jnp.int32, (R, R), 1)
    ltri = (rj < ri).astype(jnp.float32)
    local_pref = jnp.dot(ltri, S, preferred_element_type=jnp.float32)
    FF = acc_ref[0:1, :] + local_pref
    acc_ref[0:1, :] = acc_ref[0:1, :] + local_pref[R - 1 : R, :] + S[R - 1 : R, :]
    if accout_ref is not None:
        accout_ref[...] = acc_ref[...]

    valid = cols <= rows
    FF_mod = jnp.where(valid, FF, BIG)
    bits = jax.lax.bitcast_convert_type(FF_mod, jnp.int32)
    kk = jnp.minimum(KCAP, rows + 1)

    lo = jnp.zeros((R, 1), jnp.int32)
    hi = jnp.full((R, 1), BIG_BITS + 1, jnp.int32)
    for _ in range(31):
        mid = lo + ((hi - lo) >> 1)
        cnt = jnp.sum((bits <= mid).astype(jnp.int32), axis=1, keepdims=True)
        ge = cnt >= kk
        lo = jnp.where(ge, lo, mid + 1)
        hi = jnp.where(ge, mid, hi)
    t = lo

    less = bits < t
    nless = jnp.sum(less.astype(jnp.int32), axis=1, keepdims=True)
    eq = bits == t
    pc = eq.astype(jnp.int32)
    shift = 1
    while shift < width:
        rolled = jnp.roll(pc, shift, axis=1)
        pc = pc + jnp.where(cols >= shift, rolled, 0)
        shift *= 2
    keep = less | (eq & (pc <= (kk - nless))) | (cols == rows)

    neg_ff = -FF
    ys = []
    for h in range(H):
        if h == 0:
            sh = att0
        else:
            qh = q_ref[:, h * HD : (h + 1) * HD].astype(jnp.bfloat16)
            kh = k_ref[:, h * HD : (h + 1) * HD].astype(jnp.bfloat16)
            sh = jax.lax.dot_general(
                qh, kh, (((1,), (1,)), ((), ())),
                preferred_element_type=jnp.float32,
            ) * SCALE
        a = jnp.where(keep, sh + neg_ff, NEG)
        m = jnp.max(a, axis=1, keepdims=True)
        e = jnp.exp(a - m)
        denom = jnp.sum(e, axis=1, keepdims=True)
        vh = v_ref[:, h * HD : (h + 1) * HD].astype(jnp.bfloat16)
        yh = jnp.dot(e.astype(jnp.bfloat16), vh, preferred_element_type=jnp.float32)
        ys.append(yh / denom)
    y = jnp.concatenate(ys, axis=1)

    o_ref[...] = (
        jnp.dot(
            y.astype(jnp.bfloat16),
            wp_ref[...].astype(jnp.bfloat16),
            preferred_element_type=jnp.float32,
        )
        + bp_ref[...]
    )


@jax.jit
def kernel(x, W_attn, b_attn, W_proj, b_proj):
    x2 = x.reshape(T, C)

    qkv = pl.pallas_call(
        _qkv_kernel,
        grid=(T // R,),
        in_specs=[
            pl.BlockSpec((R, C), lambda i: (i, 0)),
            pl.BlockSpec((C, 3 * C), lambda i: (0, 0)),
            pl.BlockSpec((1, 3 * C), lambda i: (0, 0)),
        ],
        out_specs=pl.BlockSpec((R, 3 * C), lambda i: (i, 0)),
        out_shape=jax.ShapeDtypeStruct((T, 3 * C), jnp.float32),
    )(x2, W_attn.T, b_attn.reshape(1, 3 * C))

    q, k, v = jnp.split(qkv, 3, axis=1)
    wp = W_proj.T
    bp = b_proj.reshape(1, C)

    y1, acc1 = pl.pallas_call(
        functools.partial(_attn_body, W1, 0),
        grid=(NB1,),
        in_specs=[
            pl.BlockSpec((R, C), lambda i: (i, 0)),
            pl.BlockSpec((W1, C), lambda i: (0, 0)),
            pl.BlockSpec((W1, C), lambda i: (0, 0)),
            pl.BlockSpec((C, C), lambda i: (0, 0)),
            pl.BlockSpec((1, C), lambda i: (0, 0)),
        ],
        out_specs=[
            pl.BlockSpec((R, C), lambda i: (i, 0)),
            pl.BlockSpec((8, W1), lambda i: (0, 0)),
        ],
        out_shape=[
            jax.ShapeDtypeStruct((W1, C), jnp.float32),
            jax.ShapeDtypeStruct((8, W1), jnp.float32),
        ],
        scratch_shapes=[pltpu.VMEM((8, W1), jnp.float32)],
        compiler_params=pltpu.CompilerParams(
            dimension_semantics=("arbitrary",),
        ),
    )(q, k, v, wp, bp)

    y2 = pl.pallas_call(
        functools.partial(_attn_body, T, NB1),
        grid=(T // R - NB1,),
        in_specs=[
            pl.BlockSpec((R, C), lambda i: (i + NB1, 0)),
            pl.BlockSpec((T, C), lambda i: (0, 0)),
            pl.BlockSpec((T, C), lambda i: (0, 0)),
            pl.BlockSpec((C, C), lambda i: (0, 0)),
            pl.BlockSpec((1, C), lambda i: (0, 0)),
            pl.BlockSpec((8, W1), lambda i: (0, 0)),
        ],
        out_specs=pl.BlockSpec((R, C), lambda i: (i, 0)),
        out_shape=jax.ShapeDtypeStruct((T - W1, C), jnp.float32),
        scratch_shapes=[pltpu.VMEM((8, T), jnp.float32)],
        compiler_params=pltpu.CompilerParams(
            dimension_semantics=("arbitrary",),
        ),
    )(q, k, v, wp, bp, acc1)

    y = jnp.concatenate([y1, y2], axis=0)
    return y.reshape(1, T, C)


# speedup vs baseline: 1.0363x; 1.0363x over previous
import functools

import jax
import jax.numpy as jnp
from jax.experimental import pallas as pl
from jax.experimental.pallas import tpu as pltpu

T = 2048
C = 768
H = 12
HD = C // H
R = 256
NB1 = 4
W1 = NB1 * R
NEG = -1e30
SCALE = 1.0 / (HD ** 0.5)
BUDGET = max(1, int(T * 0.2))
KCAP = BUDGET - 1
BIG = 1e30
BIG_BITS = 0x7149F2CA


def _qkv_kernel(x_ref, w_ref, b_ref, o_ref):
    o_ref[...] = (
        jnp.dot(x_ref[...], w_ref[...], preferred_element_type=jnp.float32)
        + b_ref[...]
    )


def _attn_body(width, row_off, q_ref, k_ref, v_ref, wp_ref, bp_ref, *rest):
    if row_off == 0:
        o_ref, accout_ref, acc_ref = rest
        accin_ref = None
    else:
        accin_ref, o_ref, acc_ref = rest
        accout_ref = None

    pid = pl.program_id(0)

    @pl.when(pid == 0)
    def _init():
        if accin_ref is None:
            acc_ref[...] = jnp.zeros_like(acc_ref)
        else:
            acc_ref[...] = jnp.concatenate(
                [accin_ref[...], jnp.zeros((8, width - W1), jnp.float32)], axis=1
            )

    i0 = (pid + row_off) * R
    rows = i0 + jax.lax.broadcasted_iota(jnp.int32, (R, 1), 0)
    cols = jax.lax.broadcasted_iota(jnp.int32, (R, width), 1)

    q0 = q_ref[:, 0:HD]
    k0 = k_ref[:, 0:HD]
    att0 = jax.lax.dot_general(
        q0, k0, (((1,), (1,)), ((), ())), preferred_element_type=jnp.float32
    ) * SCALE
    s_mask = (cols < rows) & (cols > 0)
    S = jnp.where(s_mask, jnp.maximum(att0, 0.0), 0.0)

    ri = jax.lax.broadcasted_iota(jnp.int32, (R, R), 0)
    rj = jax.lax.broadcasted_iota(jnp.int32, (R, R), 1)
    ltri = (rj < ri).astype(jnp.float32)
    local_pref = jnp.dot(ltri, S, preferred_element_type=jnp.float32)
    FF = acc_ref[0:1, :] + local_pref
    acc_ref[0:1, :] = acc_ref[0:1, :] + local_pref[R - 1 : R, :] + S[R - 1 : R, :]
    if accout_ref is not None:
        accout_ref[...] = acc_ref[...]

    valid = cols <= rows
    FF_mod = jnp.where(valid, FF, BIG)
    bits = jax.lax.bitcast_convert_type(FF_mod, jnp.int32)
    kk = jnp.minimum(KCAP, rows + 1)

    lo = jnp.zeros((R, 1), jnp.int32)
    hi = jnp.full((R, 1), BIG_BITS + 1, jnp.int32)
    for _ in range(31):
        mid = lo + ((hi - lo) >> 1)
        cnt = jnp.sum((bits <= mid).astype(jnp.int32), axis=1, keepdims=True)
        ge = cnt >= kk
        lo = jnp.where(ge, lo, mid + 1)
        hi = jnp.where(ge, mid, hi)
    t = lo

    less = bits < t
    nless = jnp.sum(less.astype(jnp.int32), axis=1, keepdims=True)
    eq = bits == t
    pc = eq.astype(jnp.int32)
    shift = 1
    while shift < width:
        rolled = jnp.roll(pc, shift, axis=1)
        pc = pc + jnp.where(cols >= shift, rolled, 0)
        shift *= 2
    keep = less | (eq & (pc <= (kk - nless))) | (cols == rows)

    neg_ff = -FF
    ones_col = jnp.ones((width, 8), jnp.bfloat16)
    ys = []
    for h in range(H):
        if h == 0:
            sh = att0
        else:
            qh = q_ref[:, h * HD : (h + 1) * HD].astype(jnp.bfloat16)
            kh = k_ref[:, h * HD : (h + 1) * HD].astype(jnp.bfloat16)
            sh = jax.lax.dot_general(
                qh, kh, (((1,), (1,)), ((), ())),
                preferred_element_type=jnp.float32,
            ) * SCALE
        a = jnp.where(keep, sh + neg_ff, NEG)
        e = jnp.exp(a).astype(jnp.bfloat16)
        denom = jnp.dot(e, ones_col, preferred_element_type=jnp.float32)[:, 0:1]
        vh = v_ref[:, h * HD : (h + 1) * HD].astype(jnp.bfloat16)
        yh = jnp.dot(e, vh, preferred_element_type=jnp.float32)
        ys.append(yh / denom)
    y = jnp.concatenate(ys, axis=1)

    o_ref[...] = (
        jnp.dot(y, wp_ref[...], preferred_element_type=jnp.float32) + bp_ref[...]
    )


@jax.jit
def kernel(x, W_attn, b_attn, W_proj, b_proj):
    x2 = x.reshape(T, C)

    qkv = pl.pallas_call(
        _qkv_kernel,
        grid=(T // R,),
        in_specs=[
            pl.BlockSpec((R, C), lambda i: (i, 0)),
            pl.BlockSpec((C, 3 * C), lambda i: (0, 0)),
            pl.BlockSpec((1, 3 * C), lambda i: (0, 0)),
        ],
        out_specs=pl.BlockSpec((R, 3 * C), lambda i: (i, 0)),
        out_shape=jax.ShapeDtypeStruct((T, 3 * C), jnp.float32),
    )(x2, W_attn.T, b_attn.reshape(1, 3 * C))

    q, k, v = jnp.split(qkv, 3, axis=1)
    wp = W_proj.T
    bp = b_proj.reshape(1, C)

    y1, acc1 = pl.pallas_call(
        functools.partial(_attn_body, W1, 0),
        grid=(NB1,),
        in_specs=[
            pl.BlockSpec((R, C), lambda i: (i, 0)),
            pl.BlockSpec((W1, C), lambda i: (0, 0)),
            pl.BlockSpec((W1, C), lambda i: (0, 0)),
            pl.BlockSpec((C, C), lambda i: (0, 0)),
            pl.BlockSpec((1, C), lambda i: (0, 0)),
        ],
        out_specs=[
            pl.BlockSpec((R, C), lambda i: (i, 0)),
            pl.BlockSpec((8, W1), lambda i: (0, 0)),
        ],
        out_shape=[
            jax.ShapeDtypeStruct((W1, C), jnp.float32),
            jax.ShapeDtypeStruct((8, W1), jnp.float32),
        ],
        scratch_shapes=[pltpu.VMEM((8, W1), jnp.float32)],
        compiler_params=pltpu.CompilerParams(
            dimension_semantics=("arbitrary",),
        ),
    )(q, k, v, wp, bp)

    y2 = pl.pallas_call(
        functools.partial(_attn_body, T, NB1),
        grid=(T // R - NB1,),
        in_specs=[
            pl.BlockSpec((R, C), lambda i: (i + NB1, 0)),
            pl.BlockSpec((T, C), lambda i: (0, 0)),
            pl.BlockSpec((T, C), lambda i: (0, 0)),
            pl.BlockSpec((C, C), lambda i: (0, 0)),
            pl.BlockSpec((1, C), lambda i: (0, 0)),
            pl.BlockSpec((8, W1), lambda i: (0, 0)),
        ],
        out_specs=pl.BlockSpec((R, C), lambda i: (i, 0)),
        out_shape=jax.ShapeDtypeStruct((T - W1, C), jnp.float32),
        scratch_shapes=[pltpu.VMEM((8, T), jnp.float32)],
        compiler_params=pltpu.CompilerParams(
            dimension_semantics=("arbitrary",),
        ),
    )(q, k, v, wp, bp, acc1)

    y = jnp.concatenate([y1, y2], axis=0)
    return y.reshape(1, T, C)


# speedup vs baseline: 1.2091x; 1.1667x over previous
import functools

import jax
import jax.numpy as jnp
from jax.experimental import pallas as pl
from jax.experimental.pallas import tpu as pltpu

T = 2048
C = 768
H = 12
HD = C // H
R = 256
NB1 = 4
W1 = NB1 * R
NEG = -1e30
SCALE = 1.0 / (HD ** 0.5)
BUDGET = max(1, int(T * 0.2))
KCAP = BUDGET - 1
BIG = 1e30
BIG_BITS = 0x7149F2CA


def _qkv_kernel(x_ref, w_ref, b_ref, oq_ref, ok_ref, ov_ref):
    out = jax.lax.dot_general(
        x_ref[...], w_ref[...], (((1,), (1,)), ((), ())),
        preferred_element_type=jnp.float32,
    ) + b_ref[...]
    oq_ref[...] = out[:, 0:C]
    ok_ref[...] = out[:, C : 2 * C]
    ov_ref[...] = out[:, 2 * C : 3 * C]


def _attn_body(width, row_off, q_ref, k_ref, v_ref, wp_ref, bp_ref, *rest):
    if row_off == 0:
        o_ref, accout_ref, acc_ref = rest
        accin_ref = None
    else:
        accin_ref, o_ref, acc_ref = rest
        accout_ref = None

    pid = pl.program_id(0)

    @pl.when(pid == 0)
    def _init():
        if accin_ref is None:
            acc_ref[...] = jnp.zeros_like(acc_ref)
        else:
            acc_ref[...] = jnp.concatenate(
                [accin_ref[...], jnp.zeros((8, width - W1), jnp.float32)], axis=1
            )

    i0 = (pid + row_off) * R
    rows = i0 + jax.lax.broadcasted_iota(jnp.int32, (R, 1), 0)
    cols = jax.lax.broadcasted_iota(jnp.int32, (R, width), 1)

    q0 = q_ref[:, 0:HD]
    k0 = k_ref[:, 0:HD]
    att0 = jax.lax.dot_general(
        q0, k0, (((1,), (1,)), ((), ())), preferred_element_type=jnp.float32
    ) * SCALE
    s_mask = (cols < rows) & (cols > 0)
    S = jnp.where(s_mask, jnp.maximum(att0, 0.0), 0.0)

    ri = jax.lax.broadcasted_iota(jnp.int32, (R, R), 0)
    rj = jax.lax.broadcasted_iota(jnp.int32, (R, R), 1)
    ltri = (rj < ri).astype(jnp.float32)
    local_pref = jnp.dot(ltri, S, preferred_element_type=jnp.float32)
    FF = acc_ref[0:1, :] + local_pref
    acc_ref[0:1, :] = acc_ref[0:1, :] + local_pref[R - 1 : R, :] + S[R - 1 : R, :]
    if accout_ref is not None:
        accout_ref[...] = acc_ref[...]

    valid = cols <= rows
    FF_mod = jnp.where(valid, FF, BIG)
    bits = jax.lax.bitcast_convert_type(FF_mod, jnp.int32)
    kk = jnp.minimum(KCAP, rows + 1)

    lo = jnp.zeros((R, 1), jnp.int32)
    hi = jnp.full((R, 1), BIG_BITS + 1, jnp.int32)
    for _ in range(31):
        mid = lo + ((hi - lo) >> 1)
        cnt = jnp.sum((bits <= mid).astype(jnp.int32), axis=1, keepdims=True)
        ge = cnt >= kk
        lo = jnp.where(ge, lo, mid + 1)
        hi = jnp.where(ge, mid, hi)
    t = lo

    less = bits < t
    nless = jnp.sum(less.astype(jnp.int32), axis=1, keepdims=True)
    eq = bits == t
    pc = eq.astype(jnp.int32)
    shift = 1
    while shift < width:
        rolled = jnp.roll(pc, shift, axis=1)
        pc = pc + jnp.where(cols >= shift, rolled, 0)
        shift *= 2
    keep = less | (eq & (pc <= (kk - nless))) | (cols == rows)

    neg_ff = -FF
    ones_col = jnp.ones((width, 8), jnp.bfloat16)
    ys = []
    for h in range(H):
        if h == 0:
            sh = att0
        else:
            qh = q_ref[:, h * HD : (h + 1) * HD].astype(jnp.bfloat16)
            kh = k_ref[:, h * HD : (h + 1) * HD].astype(jnp.bfloat16)
            sh = jax.lax.dot_general(
                qh, kh, (((1,), (1,)), ((), ())),
                preferred_element_type=jnp.float32,
            ) * SCALE
        a = jnp.where(keep, sh + neg_ff, NEG)
        e = jnp.exp(a).astype(jnp.bfloat16)
        denom = jnp.dot(e, ones_col, preferred_element_type=jnp.float32)[:, 0:1]
        vh = v_ref[:, h * HD : (h + 1) * HD].astype(jnp.bfloat16)
        yh = jnp.dot(e, vh, preferred_element_type=jnp.float32)
        ys.append(yh / denom)
    y = jnp.concatenate(ys, axis=1)

    o_ref[...] = (
        jax.lax.dot_general(
            y, wp_ref[...], (((1,), (1,)), ((), ())),
            preferred_element_type=jnp.float32,
        )
        + bp_ref[...]
    )


@jax.jit
def kernel(x, W_attn, b_attn, W_proj, b_proj):
    x2 = x.reshape(T, C)

    q, k, v = pl.pallas_call(
        _qkv_kernel,
        grid=(T // R,),
        in_specs=[
            pl.BlockSpec((R, C), lambda i: (i, 0)),
            pl.BlockSpec((3 * C, C), lambda i: (0, 0)),
            pl.BlockSpec((1, 3 * C), lambda i: (0, 0)),
        ],
        out_specs=[
            pl.BlockSpec((R, C), lambda i: (i, 0)),
            pl.BlockSpec((R, C), lambda i: (i, 0)),
            pl.BlockSpec((R, C), lambda i: (i, 0)),
        ],
        out_shape=[
            jax.ShapeDtypeStruct((T, C), jnp.float32),
            jax.ShapeDtypeStruct((T, C), jnp.float32),
            jax.ShapeDtypeStruct((T, C), jnp.float32),
        ],
    )(x2, W_attn, b_attn.reshape(1, 3 * C))

    wp = W_proj
    bp = b_proj.reshape(1, C)

    y1, acc1 = pl.pallas_call(
        functools.partial(_attn_body, W1, 0),
        grid=(NB1,),
        in_specs=[
            pl.BlockSpec((R, C), lambda i: (i, 0)),
            pl.BlockSpec((W1, C), lambda i: (0, 0)),
            pl.BlockSpec((W1, C), lambda i: (0, 0)),
            pl.BlockSpec((C, C), lambda i: (0, 0)),
            pl.BlockSpec((1, C), lambda i: (0, 0)),
        ],
        out_specs=[
            pl.BlockSpec((R, C), lambda i: (i, 0)),
            pl.BlockSpec((8, W1), lambda i: (0, 0)),
        ],
        out_shape=[
            jax.ShapeDtypeStruct((W1, C), jnp.float32),
            jax.ShapeDtypeStruct((8, W1), jnp.float32),
        ],
        scratch_shapes=[pltpu.VMEM((8, W1), jnp.float32)],
        compiler_params=pltpu.CompilerParams(
            dimension_semantics=("arbitrary",),
        ),
    )(q, k, v, wp, bp)

    y2 = pl.pallas_call(
        functools.partial(_attn_body, T, NB1),
        grid=(T // R - NB1,),
        in_specs=[
            pl.BlockSpec((R, C), lambda i: (i + NB1, 0)),
            pl.BlockSpec((T, C), lambda i: (0, 0)),
            pl.BlockSpec((T, C), lambda i: (0, 0)),
            pl.BlockSpec((C, C), lambda i: (0, 0)),
            pl.BlockSpec((1, C), lambda i: (0, 0)),
            pl.BlockSpec((8, W1), lambda i: (0, 0)),
        ],
        out_specs=pl.BlockSpec((R, C), lambda i: (i, 0)),
        out_shape=jax.ShapeDtypeStruct((T - W1, C), jnp.float32),
        scratch_shapes=[pltpu.VMEM((8, T), jnp.float32)],
        compiler_params=pltpu.CompilerParams(
            dimension_semantics=("arbitrary",),
        ),
    )(q, k, v, wp, bp, acc1)

    y = jnp.concatenate([y1, y2], axis=0)
    return y.reshape(1, T, C)
